# Initial kernel scaffold; baseline (speedup 1.0000x reference)
import jax
import jax.numpy as jnp
from jax.experimental import pallas as pl


def kernel(x, W):
    raise NotImplementedError("write your pallas kernel here")


# speedup vs baseline: 2.8009x; 2.8009x over previous
import functools

import jax
import jax.numpy as jnp
from jax.experimental import pallas as pl
from jax.experimental.pallas import tpu as pltpu

_TOP_K = 8
_ALPHA = 0.001


def _gate_kernel(x_ref, wt_ref, idx_ref, wgt_ref, aux_ref,
                 cnt_acc, ssum_acc, *, nblocks, blocks_per_batch,
                 num_batches, seq_len, num_experts):
    i = pl.program_id(0)

    @pl.when(i == 0)
    def _init():
        cnt_acc[...] = jnp.zeros_like(cnt_acc)
        ssum_acc[...] = jnp.zeros_like(ssum_acc)

    logits = jnp.dot(x_ref[...], wt_ref[...],
                     preferred_element_type=jnp.float32)
    m = jnp.max(logits, axis=-1, keepdims=True)
    e = jnp.exp(logits - m)
    denom = jnp.sum(e, axis=-1, keepdims=True)
    scores = e / denom

    t = scores.shape[0]
    lane = jax.lax.broadcasted_iota(jnp.int32, (t, num_experts), 1)

    work = scores
    sel_total = jnp.zeros_like(scores)
    for k in range(_TOP_K):
        mk = jnp.max(work, axis=-1, keepdims=True)
        is_max = work == mk
        idxk = jnp.min(jnp.where(is_max, lane, num_experts),
                       axis=-1, keepdims=True)
        sel = lane == idxk
        wgt_ref[:, k:k + 1] = mk
        idx_ref[:, k:k + 1] = idxk
        work = jnp.where(sel, -1.0, work)
        sel_total = sel_total + sel.astype(jnp.float32)

    b = i // blocks_per_batch
    bhot = (jax.lax.broadcasted_iota(jnp.int32, (num_batches, 1), 0)
            == b).astype(jnp.float32)
    cnt_acc[...] += bhot * jnp.sum(sel_total, axis=0)[None, :]
    ssum_acc[...] += bhot * jnp.sum(scores, axis=0)[None, :]

    @pl.when(i == nblocks - 1)
    def _finalize():
        ce = cnt_acc[...] * (num_experts / (seq_len * _TOP_K))
        mean_scores = ssum_acc[...] * (1.0 / seq_len)
        aux = (jnp.sum(ce * mean_scores) / num_batches) * _ALPHA
        aux_ref[...] = jnp.full((1, 1), aux, dtype=jnp.float32)


@jax.jit
def kernel(x, W):
    bsz, seq_len, dim = x.shape
    num_experts = W.shape[0]
    tokens = bsz * seq_len
    hidden = x.reshape(tokens, dim)
    wt = W.T

    block_t = 1024
    nblocks = tokens // block_t
    blocks_per_batch = seq_len // block_t

    kfn = functools.partial(
        _gate_kernel,
        nblocks=nblocks,
        blocks_per_batch=blocks_per_batch,
        num_batches=bsz,
        seq_len=seq_len,
        num_experts=num_experts,
    )

    idx, wgt, aux = pl.pallas_call(
        kfn,
        grid=(nblocks,),
        in_specs=[
            pl.BlockSpec((block_t, dim), lambda i: (i, 0)),
            pl.BlockSpec((dim, num_experts), lambda i: (0, 0)),
        ],
        out_specs=[
            pl.BlockSpec((block_t, _TOP_K), lambda i: (i, 0)),
            pl.BlockSpec((block_t, _TOP_K), lambda i: (i, 0)),
            pl.BlockSpec((1, 1), lambda i: (0, 0)),
        ],
        out_shape=[
            jax.ShapeDtypeStruct((tokens, _TOP_K), jnp.int32),
            jax.ShapeDtypeStruct((tokens, _TOP_K), jnp.float32),
            jax.ShapeDtypeStruct((1, 1), jnp.float32),
        ],
        scratch_shapes=[
            pltpu.VMEM((bsz, num_experts), jnp.float32),
            pltpu.VMEM((bsz, num_experts), jnp.float32),
        ],
    )(hidden, wt)

    return idx, wgt, aux[0, 0]


# speedup vs baseline: 8.2279x; 2.9376x over previous
import functools

import jax
import jax.numpy as jnp
from jax.experimental import pallas as pl
from jax.experimental.pallas import tpu as pltpu

_TOP_K = 8
_ALPHA = 0.001


def _gate_kernel(x_ref, w_ref, idx_ref, wgt_ref, aux_ref,
                 cnt_acc, ssum_acc, *, nblocks, blocks_per_batch,
                 num_batches, seq_len, num_experts):
    i = pl.program_id(0)

    @pl.when(i == 0)
    def _init():
        cnt_acc[...] = jnp.zeros_like(cnt_acc)
        ssum_acc[...] = jnp.zeros_like(ssum_acc)

    logits = jax.lax.dot_general(
        w_ref[...], x_ref[...],
        dimension_numbers=(((1,), (1,)), ((), ())),
        preferred_element_type=jnp.float32)

    m = jnp.max(logits, axis=0, keepdims=True)
    e = jnp.exp(logits - m)
    denom = jnp.sum(e, axis=0, keepdims=True)
    scores = e * (1.0 / denom)

    t = scores.shape[1]
    eid = jax.lax.broadcasted_iota(jnp.int32, (num_experts, t), 0)

    work = scores
    wgt_rows = []
    idx_rows = []
    for _ in range(_TOP_K):
        mk = jnp.max(work, axis=0, keepdims=True)
        is_max = work == mk
        idxk = jnp.min(jnp.where(is_max, eid, num_experts),
                       axis=0, keepdims=True)
        sel = eid == idxk
        work = jnp.where(sel, -1.0, work)
        wgt_rows.append(mk)
        idx_rows.append(idxk)
    wgt_ref[...] = jnp.concatenate(wgt_rows, axis=0)
    idx_ref[...] = jnp.concatenate(idx_rows, axis=0)

    sel_cnt = jnp.sum((work < 0).astype(jnp.float32), axis=1,
                      keepdims=True)
    s_sum = jnp.sum(scores, axis=1, keepdims=True)

    b = i // blocks_per_batch
    bhot = (jax.lax.broadcasted_iota(jnp.int32, (1, num_batches), 1)
            == b).astype(jnp.float32)
    cnt_acc[...] += sel_cnt * bhot
    ssum_acc[...] += s_sum * bhot

    @pl.when(i == nblocks - 1)
    def _finalize():
        ce = cnt_acc[...] * (num_experts / (seq_len * _TOP_K))
        mean_scores = ssum_acc[...] * (1.0 / seq_len)
        aux = (jnp.sum(ce * mean_scores) / num_batches) * _ALPHA
        aux_ref[...] = jnp.full((1, 1), aux, dtype=jnp.float32)


@jax.jit
def kernel(x, W):
    bsz, seq_len, dim = x.shape
    num_experts = W.shape[0]
    tokens = bsz * seq_len
    hidden = x.reshape(tokens, dim)

    block_t = 1024
    nblocks = tokens // block_t
    blocks_per_batch = seq_len // block_t

    kfn = functools.partial(
        _gate_kernel,
        nblocks=nblocks,
        blocks_per_batch=blocks_per_batch,
        num_batches=bsz,
        seq_len=seq_len,
        num_experts=num_experts,
    )

    idx_t, wgt_t, aux = pl.pallas_call(
        kfn,
        grid=(nblocks,),
        in_specs=[
            pl.BlockSpec((block_t, dim), lambda i: (i, 0)),
            pl.BlockSpec((num_experts, dim), lambda i: (0, 0)),
        ],
        out_specs=[
            pl.BlockSpec((_TOP_K, block_t), lambda i: (0, i)),
            pl.BlockSpec((_TOP_K, block_t), lambda i: (0, i)),
            pl.BlockSpec((1, 1), lambda i: (0, 0)),
        ],
        out_shape=[
            jax.ShapeDtypeStruct((_TOP_K, tokens), jnp.int32),
            jax.ShapeDtypeStruct((_TOP_K, tokens), jnp.float32),
            jax.ShapeDtypeStruct((1, 1), jnp.float32),
        ],
        scratch_shapes=[
            pltpu.VMEM((num_experts, bsz), jnp.float32),
            pltpu.VMEM((num_experts, bsz), jnp.float32),
        ],
    )(hidden, W)

    return idx_t.T, wgt_t.T, aux[0, 0]


# speedup vs baseline: 9.7806x; 1.1887x over previous
import functools

import jax
import jax.numpy as jnp
from jax.experimental import pallas as pl
from jax.experimental.pallas import tpu as pltpu

_TOP_K = 8
_ALPHA = 0.001


def _gate_kernel(x_ref, w_ref, idx_ref, wgt_ref, aux_ref,
                 cnt_acc, ssum_acc, *, nblocks, blocks_per_batch,
                 num_batches, seq_len, num_experts):
    i = pl.program_id(0)

    @pl.when(i == 0)
    def _init():
        cnt_acc[...] = jnp.zeros_like(cnt_acc)
        ssum_acc[...] = jnp.zeros_like(ssum_acc)

    logits = jax.lax.dot_general(
        w_ref[...], x_ref[...],
        dimension_numbers=(((1,), (1,)), ((), ())),
        preferred_element_type=jnp.float32)

    m = jnp.max(logits, axis=0, keepdims=True)
    e = jnp.exp(logits - m)
    denom = jnp.sum(e, axis=0, keepdims=True)
    scores = e * (1.0 / denom)

    t = scores.shape[1]
    eid = jax.lax.broadcasted_iota(jnp.int32, (num_experts, t), 0)

    work = scores
    wgt_rows = []
    idx_rows = []
    for _ in range(_TOP_K):
        mk = jnp.max(work, axis=0, keepdims=True)
        is_max = work == mk
        idxk = jnp.min(jnp.where(is_max, eid, num_experts),
                       axis=0, keepdims=True)
        sel = eid == idxk
        work = jnp.where(sel, -1.0, work)
        wgt_rows.append(mk)
        idx_rows.append(idxk)
    wgt_ref[...] = jnp.concatenate(wgt_rows, axis=0)
    idx_ref[...] = jnp.concatenate(idx_rows, axis=0)

    sel_cnt = jnp.sum((work < 0).astype(jnp.float32), axis=1,
                      keepdims=True)
    s_sum = jnp.sum(scores, axis=1, keepdims=True)

    b = i // blocks_per_batch
    bhot = (jax.lax.broadcasted_iota(jnp.int32, (1, num_batches), 1)
            == b).astype(jnp.float32)
    cnt_acc[...] += sel_cnt * bhot
    ssum_acc[...] += s_sum * bhot

    @pl.when(i == nblocks - 1)
    def _finalize():
        ce = cnt_acc[...] * (num_experts / (seq_len * _TOP_K))
        mean_scores = ssum_acc[...] * (1.0 / seq_len)
        aux = (jnp.sum(ce * mean_scores) / num_batches) * _ALPHA
        aux_ref[...] = jnp.full((1, 1), aux, dtype=jnp.float32)


@jax.jit
def kernel(x, W):
    bsz, seq_len, dim = x.shape
    num_experts = W.shape[0]
    tokens = bsz * seq_len
    hidden = x.reshape(tokens, dim)

    block_t = 2048
    nblocks = tokens // block_t
    blocks_per_batch = seq_len // block_t

    kfn = functools.partial(
        _gate_kernel,
        nblocks=nblocks,
        blocks_per_batch=blocks_per_batch,
        num_batches=bsz,
        seq_len=seq_len,
        num_experts=num_experts,
    )

    idx_t, wgt_t, aux = pl.pallas_call(
        kfn,
        grid=(nblocks,),
        in_specs=[
            pl.BlockSpec((block_t, dim), lambda i: (i, 0)),
            pl.BlockSpec((num_experts, dim), lambda i: (0, 0)),
        ],
        out_specs=[
            pl.BlockSpec((_TOP_K, block_t), lambda i: (0, i)),
            pl.BlockSpec((_TOP_K, block_t), lambda i: (0, i)),
            pl.BlockSpec((1, 1), lambda i: (0, 0)),
        ],
        out_shape=[
            jax.ShapeDtypeStruct((_TOP_K, tokens), jnp.int32),
            jax.ShapeDtypeStruct((_TOP_K, tokens), jnp.float32),
            jax.ShapeDtypeStruct((1, 1), jnp.float32),
        ],
        scratch_shapes=[
            pltpu.VMEM((num_experts, bsz), jnp.float32),
            pltpu.VMEM((num_experts, bsz), jnp.float32),
        ],
    )(hidden, W)

    return idx_t.T, wgt_t.T, aux[0, 0]


# speedup vs baseline: 10.7102x; 1.0950x over previous
import functools

import jax
import jax.numpy as jnp
from jax.experimental import pallas as pl
from jax.experimental.pallas import tpu as pltpu

_TOP_K = 8
_ALPHA = 0.001


def _gate_kernel(x_ref, w_ref, idx_ref, wgt_ref, aux_ref,
                 cnt_acc, ssum_acc, *, nblocks, blocks_per_batch,
                 num_batches, seq_len, num_experts):
    i = pl.program_id(0)

    @pl.when(i == 0)
    def _init():
        cnt_acc[...] = jnp.zeros_like(cnt_acc)
        ssum_acc[...] = jnp.zeros_like(ssum_acc)

    logits = jax.lax.dot_general(
        w_ref[...], x_ref[...],
        dimension_numbers=(((1,), (1,)), ((), ())),
        preferred_element_type=jnp.float32)

    m = jnp.max(logits, axis=0, keepdims=True)
    e = jnp.exp(logits - m)
    denom = jnp.sum(e, axis=0, keepdims=True)
    scores = e * (1.0 / denom)

    t = scores.shape[1]
    eid = jax.lax.broadcasted_iota(jnp.int32, (num_experts, t), 0)

    work = scores
    wgt_rows = []
    idx_rows = []
    for _ in range(_TOP_K):
        mk = jnp.max(work, axis=0, keepdims=True)
        is_max = work == mk
        idxk = jnp.min(jnp.where(is_max, eid, num_experts),
                       axis=0, keepdims=True)
        sel = eid == idxk
        work = jnp.where(sel, -1.0, work)
        wgt_rows.append(mk)
        idx_rows.append(idxk)
    wgt_ref[...] = jnp.concatenate(wgt_rows, axis=0)
    idx_ref[...] = jnp.concatenate(idx_rows, axis=0)

    sel_cnt = jnp.sum((work < 0).astype(jnp.float32), axis=1,
                      keepdims=True)
    s_sum = jnp.sum(scores, axis=1, keepdims=True)

    b = i // blocks_per_batch
    bhot = (jax.lax.broadcasted_iota(jnp.int32, (1, num_batches), 1)
            == b).astype(jnp.float32)
    cnt_acc[...] += sel_cnt * bhot
    ssum_acc[...] += s_sum * bhot

    @pl.when(i == nblocks - 1)
    def _finalize():
        ce = cnt_acc[...] * (num_experts / (seq_len * _TOP_K))
        mean_scores = ssum_acc[...] * (1.0 / seq_len)
        aux = (jnp.sum(ce * mean_scores) / num_batches) * _ALPHA
        aux_ref[...] = jnp.full((1, 1), aux, dtype=jnp.float32)


@jax.jit
def kernel(x, W):
    bsz, seq_len, dim = x.shape
    num_experts = W.shape[0]
    tokens = bsz * seq_len
    hidden = x.reshape(tokens, dim)

    block_t = 4096
    nblocks = tokens // block_t
    blocks_per_batch = seq_len // block_t

    kfn = functools.partial(
        _gate_kernel,
        nblocks=nblocks,
        blocks_per_batch=blocks_per_batch,
        num_batches=bsz,
        seq_len=seq_len,
        num_experts=num_experts,
    )

    idx_t, wgt_t, aux = pl.pallas_call(
        kfn,
        grid=(nblocks,),
        in_specs=[
            pl.BlockSpec((block_t, dim), lambda i: (i, 0)),
            pl.BlockSpec((num_experts, dim), lambda i: (0, 0)),
        ],
        out_specs=[
            pl.BlockSpec((_TOP_K, block_t), lambda i: (0, i)),
            pl.BlockSpec((_TOP_K, block_t), lambda i: (0, i)),
            pl.BlockSpec((1, 1), lambda i: (0, 0)),
        ],
        out_shape=[
            jax.ShapeDtypeStruct((_TOP_K, tokens), jnp.int32),
            jax.ShapeDtypeStruct((_TOP_K, tokens), jnp.float32),
            jax.ShapeDtypeStruct((1, 1), jnp.float32),
        ],
        scratch_shapes=[
            pltpu.VMEM((num_experts, bsz), jnp.float32),
            pltpu.VMEM((num_experts, bsz), jnp.float32),
        ],
    )(hidden, W)

    return idx_t.T, wgt_t.T, aux[0, 0]
